# Initial kernel scaffold; baseline (speedup 1.0000x reference)
import jax
import jax.numpy as jnp
from jax import lax
from jax.experimental import pallas as pl
from jax.experimental.pallas import tpu as pltpu
from jax.experimental.pallas import tpu_sc as plsc

N = 10000
D = 128
E = 320000
M2E = 2 * E
P = 50000
NPAD = 10240
PPAD = 50176
NW = 32
NPT = NPAD // NW
PW = PPAD // NW
C = 1024
T = 512
SEL = T + C + 16
NEG = -3.0e38
F16 = D // 16

_mesh = plsc.VectorSubcoreMesh(core_axis_name="c", subcore_axis_name="s")


def _mm_y_body(x_ref, w_ref, o_ref):
    o_ref[...] = jnp.dot(x_ref[...], w_ref[...],
                         preferred_element_type=jnp.float32)


def _mm_y(xp, wT):
    BM = 1024
    return pl.pallas_call(
        _mm_y_body,
        grid=(NPAD // BM,),
        in_specs=[
            pl.BlockSpec((BM, D), lambda i: (i, 0)),
            pl.BlockSpec((D, D), lambda i: (0, 0)),
        ],
        out_specs=pl.BlockSpec((BM, D), lambda i: (i, 0)),
        out_shape=jax.ShapeDtypeStruct((NPAD, D), jnp.float32),
    )(xp, wT)


def _mm_h_body(x_ref, agg_ref, wa_ref, wb_ref, bf_ref, wn_ref, h_ref, y2_ref):
    h = jnp.dot(x_ref[...], wa_ref[...], preferred_element_type=jnp.float32)
    h += jnp.dot(agg_ref[...], wb_ref[...], preferred_element_type=jnp.float32)
    h = jnp.maximum(h + bf_ref[...], 0.0)
    h_ref[...] = h
    y2_ref[...] = jnp.dot(h, wn_ref[...], preferred_element_type=jnp.float32)


def _mm_h(xp, agg, waT, wbT, bf, wnT):
    BM = 1024
    full = pl.BlockSpec((D, D), lambda i: (0, 0))
    row = pl.BlockSpec((BM, D), lambda i: (i, 0))
    return pl.pallas_call(
        _mm_h_body,
        grid=(NPAD // BM,),
        in_specs=[row, row, full, full,
                  pl.BlockSpec((1, D), lambda i: (0, 0)), full],
        out_specs=(row, row),
        out_shape=(jax.ShapeDtypeStruct((NPAD, D), jnp.float32),
                   jax.ShapeDtypeStruct((NPAD, D), jnp.float32)),
    )(xp, agg, waT, wbT, bf, wnT)


def _sc_layer_body(y_hbm, tgt_hbm, src_hbm, scl_hbm, bp_hbm, agg_hbm,
                   tgt_c, src_c, scl_c, sel_src, sel_tgt, sel_scl,
                   rows, acc, bp_v, sem):
    wid = lax.axis_index("s") * 2 + lax.axis_index("c")
    my_base = wid * NPT
    neg = jnp.full((16,), NEG, jnp.float32)

    def init_row(i, _):
        for f in range(F16):
            acc[i, pl.ds(f * 16, 16)] = neg
        return 0
    lax.fori_loop(0, NPT + 1, init_row, 0)

    pltpu.sync_copy(bp_hbm, bp_v)

    def acc_batch(m, _):
        t_l = sel_tgt[m]
        s = sel_scl[m]
        for f in range(F16):
            sl = pl.ds(f * 16, 16)
            acc[t_l, sl] = jnp.maximum(acc[t_l, sl], rows[m, sl] * s)
        return 0

    def chunk_body(ci, cursor):
        base = ci * C
        pltpu.sync_copy(tgt_hbm.at[pl.ds(base, C)], tgt_c)
        pltpu.sync_copy(src_hbm.at[pl.ds(base, C)], src_c)
        pltpu.sync_copy(scl_hbm.at[pl.ds(base, C)], scl_c)

        def gbody(g, cur):
            sl = pl.ds(g * 16, 16)
            tg = tgt_c[sl]
            mask = (tg >= my_base) & (tg < my_base + NPT)
            plsc.store_compressed(sel_src.at[pl.ds(cur, 16)], src_c[sl],
                                  mask=mask)
            plsc.store_compressed(sel_tgt.at[pl.ds(cur, 16)], tg - my_base,
                                  mask=mask)
            plsc.store_compressed(sel_scl.at[pl.ds(cur, 16)], scl_c[sl],
                                  mask=mask)
            cnt = jnp.max(plsc.all_reduce_population_count(mask))
            return cur + cnt
        cursor = lax.fori_loop(0, C // 16, gbody, cursor)

        def flush(cur):
            pltpu.async_copy(y_hbm.at[sel_src.at[pl.ds(0, T)]], rows,
                             sem).wait()
            lax.fori_loop(0, T, acc_batch, 0)
            nrem = cur - T
            ng = (nrem + 15) // 16

            def mv(j, _):
                dd = pl.ds(j * 16, 16)
                ss = pl.ds(T + j * 16, 16)
                sel_src[dd] = sel_src[ss]
                sel_tgt[dd] = sel_tgt[ss]
                sel_scl[dd] = sel_scl[ss]
                return 0
            lax.fori_loop(0, ng, mv, 0)
            return nrem
        cursor = lax.while_loop(lambda cur: cur >= T, flush, cursor)
        return cursor

    cursor = lax.fori_loop(0, M2E // C, chunk_body, 0)

    sel_src[pl.ds(cursor, 16)] = jnp.zeros((16,), jnp.int32)
    sel_tgt[pl.ds(cursor, 16)] = jnp.full((16,), NPT, jnp.int32)
    sel_scl[pl.ds(cursor, 16)] = jnp.zeros((16,), jnp.float32)
    ng = (cursor + 15) // 16

    def drain(g, _):
        pltpu.async_copy(y_hbm.at[sel_src.at[pl.ds(g * 16, 16)]],
                         rows.at[pl.ds(0, 16)], sem).wait()

        def acc_one(l, _):
            t_l = sel_tgt[g * 16 + l]
            s = sel_scl[g * 16 + l]
            for f in range(F16):
                sl = pl.ds(f * 16, 16)
                acc[t_l, sl] = jnp.maximum(acc[t_l, sl], rows[l, sl] * s)
            return 0
        lax.fori_loop(0, 16, acc_one, 0)
        return 0
    lax.fori_loop(0, ng, drain, 0)

    def ep_row(i, _):
        for f in range(F16):
            sl = pl.ds(f * 16, 16)
            acc[i, sl] = jnp.maximum(acc[i, sl] + bp_v[sl], 0.0)
        return 0
    lax.fori_loop(0, NPT, ep_row, 0)
    pltpu.sync_copy(acc.at[pl.ds(0, NPT)], agg_hbm.at[pl.ds(my_base, NPT)])


_sc_layer = pl.kernel(
    _sc_layer_body,
    out_type=jax.ShapeDtypeStruct((NPAD, D), jnp.float32),
    mesh=_mesh,
    scratch_types=[
        pltpu.VMEM((C,), jnp.int32),
        pltpu.VMEM((C,), jnp.int32),
        pltpu.VMEM((C,), jnp.float32),
        pltpu.VMEM((SEL,), jnp.int32),
        pltpu.VMEM((SEL,), jnp.int32),
        pltpu.VMEM((SEL,), jnp.float32),
        pltpu.VMEM((T, D), jnp.float32),
        pltpu.VMEM((NPT + 1, D), jnp.float32),
        pltpu.VMEM((D,), jnp.float32),
        pltpu.SemaphoreType.DMA,
    ],
)


def _sc_pred_body(t16_hbm, p0_hbm, p1_hbm, bias_hbm, ew_hbm, ep_hbm,
                  p0_v, p1_v, rowsA, rowsB, ew_v, ep_v, bias_v, sem):
    wid = lax.axis_index("s") * 2 + lax.axis_index("c")
    base = wid * PW
    pltpu.sync_copy(p0_hbm.at[pl.ds(base, PW)], p0_v)
    pltpu.sync_copy(p1_hbm.at[pl.ds(base, PW)], p1_v)
    pltpu.sync_copy(bias_hbm, bias_v)
    pltpu.async_copy(t16_hbm.at[p0_v], rowsA, sem).wait()
    pltpu.async_copy(t16_hbm.at[p1_v], rowsB, sem).wait()
    bwp_s = bias_v[0]
    bep_s = bias_v[1]
    lane = lax.iota(jnp.int32, 16)
    c0 = jnp.zeros((16,), jnp.int32)
    c1i = jnp.full((16,), 1, jnp.int32)
    c2i = jnp.full((16,), 2, jnp.int32)
    c3i = jnp.full((16,), 3, jnp.int32)

    def g(gi, _):
        ridx = lane + gi * 16
        a0 = plsc.load_gather(rowsA, [ridx, c0])
        a2 = plsc.load_gather(rowsA, [ridx, c2i])
        b1 = plsc.load_gather(rowsB, [ridx, c1i])
        b3 = plsc.load_gather(rowsB, [ridx, c3i])
        ew_v[pl.ds(gi * 16, 16)] = jnp.maximum(a0 + b1 + bwp_s, 0.0)
        ep_v[pl.ds(gi * 16, 16)] = a2 + b3 + bep_s
        return 0
    lax.fori_loop(0, PW // 16, g, 0)
    pltpu.sync_copy(ew_v, ew_hbm.at[pl.ds(base, PW)])
    pltpu.sync_copy(ep_v, ep_hbm.at[pl.ds(base, PW)])


_sc_pred = pl.kernel(
    _sc_pred_body,
    out_type=(jax.ShapeDtypeStruct((PPAD,), jnp.float32),
              jax.ShapeDtypeStruct((PPAD,), jnp.float32)),
    mesh=_mesh,
    scratch_types=[
        pltpu.VMEM((PW,), jnp.int32),
        pltpu.VMEM((PW,), jnp.int32),
        pltpu.VMEM((PW, 16), jnp.float32),
        pltpu.VMEM((PW, 16), jnp.float32),
        pltpu.VMEM((PW,), jnp.float32),
        pltpu.VMEM((PW,), jnp.float32),
        pltpu.VMEM((16,), jnp.float32),
        pltpu.SemaphoreType.DMA,
    ],
)


@jax.jit
def _run(x, prediction_edges, message_edges, message_edgewt,
         W1p, b1p, W1f, b1f, c1, W2p, b2p, W2f, b2f, c2, Wwp, bwp, Wep, bep):
    xp = jnp.zeros((NPAD, D), jnp.float32).at[:N].set(x)
    src = message_edges[0].astype(jnp.int32)
    dst = message_edges[1].astype(jnp.int32)
    big = jnp.int32(2 ** 30)
    loop = src == dst
    tgt_all = jnp.concatenate([jnp.where(loop, big, dst),
                               jnp.where(loop, big, src)])
    src_all = jnp.concatenate([src, dst])
    s1 = 1.0 + c1 * message_edgewt
    s2 = 1.0 + c2 * message_edgewt
    s1_all = jnp.concatenate([s1, s1])
    s2_all = jnp.concatenate([s2, s2])

    y1 = _mm_y(xp, W1p.T)
    agg1 = _sc_layer(y1, tgt_all, src_all, s1_all, b1p)
    h1, y2 = _mm_h(xp, agg1, W1f[:, :D].T, W1f[:, D:].T,
                   b1f.reshape(1, D), W2p.T)
    agg2 = _sc_layer(y2, tgt_all, src_all, s2_all, b2p)

    V = jnp.zeros((D, D), jnp.float32)
    V = V.at[:, 0].set(Wwp[0, :D]).at[:, 1].set(Wwp[0, D:])
    V = V.at[:, 2].set(Wep[0, :D]).at[:, 3].set(Wep[0, D:])
    _, t = _mm_h(h1, agg2, W2f[:, :D].T, W2f[:, D:].T,
                 b2f.reshape(1, D), V)
    t16 = t[:, :16]

    p0 = jnp.zeros((PPAD,), jnp.int32).at[:P].set(
        prediction_edges[0].astype(jnp.int32))
    p1 = jnp.zeros((PPAD,), jnp.int32).at[:P].set(
        prediction_edges[1].astype(jnp.int32))
    bias16 = jnp.zeros((16,), jnp.float32).at[0].set(bwp[0]).at[1].set(bep[0])
    ew, ep = _sc_pred(t16, p0, p1, bias16)
    return ew[:P, None], ep[:P, None]


def kernel(x, prediction_edges, message_edges, message_edgewt,
           W1p, b1p, W1f, b1f, c1, W2p, b2p, W2f, b2f, c2,
           Wwp, bwp, Wep, bep):
    return _run(x, prediction_edges, message_edges, message_edgewt,
                W1p, b1p, W1f, b1f, c1, W2p, b2p, W2f, b2f, c2,
                Wwp, bwp, Wep, bep)


# speedup vs baseline: 1.1508x; 1.1508x over previous
import jax
import jax.numpy as jnp
from jax import lax
from jax.experimental import pallas as pl
from jax.experimental.pallas import tpu as pltpu
from jax.experimental.pallas import tpu_sc as plsc

N = 10000
D = 128
E = 320000
M2E = 2 * E
P = 50000
NPAD = 10240
PPAD = 50176
NW = 32
NPT = NPAD // NW
PW = PPAD // NW
C = 1024
T = 512
SEL = T + C + 16
NEG = -3.0e38
F16 = D // 16

_mesh = plsc.VectorSubcoreMesh(core_axis_name="c", subcore_axis_name="s")


def _mm_y_body(x_ref, w_ref, o_ref):
    o_ref[...] = jnp.dot(x_ref[...], w_ref[...],
                         precision=lax.Precision.HIGHEST,
                         preferred_element_type=jnp.float32)


def _mm_y(xp, wT):
    BM = 1024
    return pl.pallas_call(
        _mm_y_body,
        grid=(NPAD // BM,),
        in_specs=[
            pl.BlockSpec((BM, D), lambda i: (i, 0)),
            pl.BlockSpec((D, D), lambda i: (0, 0)),
        ],
        out_specs=pl.BlockSpec((BM, D), lambda i: (i, 0)),
        out_shape=jax.ShapeDtypeStruct((NPAD, D), jnp.float32),
    )(xp, wT)


def _mm_h_body(x_ref, agg_ref, wa_ref, wb_ref, bf_ref, wn_ref, h_ref, y2_ref):
    hp = lax.Precision.HIGHEST
    h = jnp.dot(x_ref[...], wa_ref[...], precision=hp,
                preferred_element_type=jnp.float32)
    h += jnp.dot(agg_ref[...], wb_ref[...], precision=hp,
                 preferred_element_type=jnp.float32)
    h = jnp.maximum(h + bf_ref[...], 0.0)
    h_ref[...] = h
    y2_ref[...] = jnp.dot(h, wn_ref[...], precision=hp,
                          preferred_element_type=jnp.float32)


def _mm_h(xp, agg, waT, wbT, bf, wnT):
    BM = 1024
    full = pl.BlockSpec((D, D), lambda i: (0, 0))
    row = pl.BlockSpec((BM, D), lambda i: (i, 0))
    return pl.pallas_call(
        _mm_h_body,
        grid=(NPAD // BM,),
        in_specs=[row, row, full, full,
                  pl.BlockSpec((1, D), lambda i: (0, 0)), full],
        out_specs=(row, row),
        out_shape=(jax.ShapeDtypeStruct((NPAD, D), jnp.float32),
                   jax.ShapeDtypeStruct((NPAD, D), jnp.float32)),
    )(xp, agg, waT, wbT, bf, wnT)


def _sc_layer_body(y_hbm, tgt_hbm, src_hbm, scl_hbm, bp_hbm, agg_hbm,
                   tgt_c, src_c, scl_c, sel_src, sel_tgt, sel_scl,
                   rows, acc, bp_v, sem):
    wid = lax.axis_index("s") * 2 + lax.axis_index("c")
    my_base = wid * NPT
    neg = jnp.full((16,), NEG, jnp.float32)

    def init_row(i, _):
        for f in range(F16):
            acc[i, pl.ds(f * 16, 16)] = neg
        return 0
    lax.fori_loop(0, NPT + 1, init_row, 0)

    pltpu.sync_copy(bp_hbm, bp_v)

    def acc_batch(m, _):
        t_l = sel_tgt[pl.ds(m, 16)][0]
        s = sel_scl[pl.ds(m, 16)][0]
        for f in range(F16):
            sl = pl.ds(f * 16, 16)
            acc[t_l, sl] = jnp.maximum(acc[t_l, sl], rows[m, sl] * s)
        return 0

    def chunk_body(ci, cursor):
        base = ci * C
        pltpu.sync_copy(tgt_hbm.at[pl.ds(base, C)], tgt_c)
        pltpu.sync_copy(src_hbm.at[pl.ds(base, C)], src_c)
        pltpu.sync_copy(scl_hbm.at[pl.ds(base, C)], scl_c)

        def gbody(g, cur):
            sl = pl.ds(g * 16, 16)
            tg = tgt_c[sl]
            mask = (tg >= my_base) & (tg < my_base + NPT)
            pos = plsc.cumsum(mask.astype(jnp.int32)) - 1
            idx = jnp.where(mask, cur + pos, SEL - 1)
            plsc.store_scatter(sel_src, [idx], src_c[sl])
            plsc.store_scatter(sel_tgt, [idx], tg - my_base)
            plsc.store_scatter(sel_scl, [idx], scl_c[sl])
            cnt = jnp.max(pos) + 1
            return cur + cnt
        cursor = lax.fori_loop(0, C // 16, gbody, cursor)

        def flush(cur):
            pltpu.async_copy(y_hbm.at[sel_src.at[pl.ds(0, T)]], rows,
                             sem).wait()
            lax.fori_loop(0, T, acc_batch, 0)
            nrem = cur - T
            ng = (nrem + 15) // 16

            def mv(j, _):
                dd = pl.ds(j * 16, 16)
                ss = pl.ds(T + j * 16, 16)
                sel_src[dd] = sel_src[ss]
                sel_tgt[dd] = sel_tgt[ss]
                sel_scl[dd] = sel_scl[ss]
                return 0
            lax.fori_loop(0, ng, mv, 0)
            return nrem
        cursor = lax.while_loop(lambda cur: cur >= T, flush, cursor)
        return cursor

    cursor = lax.fori_loop(0, M2E // C, chunk_body, 0)

    sel_src[pl.ds(cursor, 16)] = jnp.zeros((16,), jnp.int32)
    sel_tgt[pl.ds(cursor, 16)] = jnp.full((16,), NPT, jnp.int32)
    sel_scl[pl.ds(cursor, 16)] = jnp.zeros((16,), jnp.float32)
    ng = (cursor + 15) // 16

    def drain(g, _):
        pltpu.async_copy(y_hbm.at[sel_src.at[pl.ds(g * 16, 16)]],
                         rows.at[pl.ds(0, 16)], sem).wait()

        def acc_one(l, _):
            t_l = sel_tgt[pl.ds(g * 16 + l, 16)][0]
            s = sel_scl[pl.ds(g * 16 + l, 16)][0]
            for f in range(F16):
                sl = pl.ds(f * 16, 16)
                acc[t_l, sl] = jnp.maximum(acc[t_l, sl], rows[l, sl] * s)
            return 0
        lax.fori_loop(0, 16, acc_one, 0)
        return 0
    lax.fori_loop(0, ng, drain, 0)

    def ep_row(i, _):
        for f in range(F16):
            sl = pl.ds(f * 16, 16)
            acc[i, sl] = jnp.maximum(acc[i, sl] + bp_v[sl], 0.0)
        return 0
    lax.fori_loop(0, NPT, ep_row, 0)
    pltpu.sync_copy(acc.at[pl.ds(0, NPT)], agg_hbm.at[pl.ds(my_base, NPT)])


_sc_layer = pl.kernel(
    _sc_layer_body,
    out_type=jax.ShapeDtypeStruct((NPAD, D), jnp.float32),
    mesh=_mesh,
    compiler_params=pltpu.CompilerParams(needs_layout_passes=False),
    scratch_types=[
        pltpu.VMEM((C,), jnp.int32),
        pltpu.VMEM((C,), jnp.int32),
        pltpu.VMEM((C,), jnp.float32),
        pltpu.VMEM((SEL,), jnp.int32),
        pltpu.VMEM((SEL,), jnp.int32),
        pltpu.VMEM((SEL,), jnp.float32),
        pltpu.VMEM((T, D), jnp.float32),
        pltpu.VMEM((NPT + 1, D), jnp.float32),
        pltpu.VMEM((D,), jnp.float32),
        pltpu.SemaphoreType.DMA,
    ],
)


def _sc_pred_body(t4_hbm, p0_hbm, p1_hbm, bias_hbm, ew_hbm, ep_hbm,
                  c0_v, c1_v, c2_v, c3_v, p0_v, p1_v, ew_v, ep_v, bias_v,
                  sem):
    wid = lax.axis_index("s") * 2 + lax.axis_index("c")
    base = wid * PW
    pltpu.sync_copy(t4_hbm.at[0], c0_v)
    pltpu.sync_copy(t4_hbm.at[1], c1_v)
    pltpu.sync_copy(t4_hbm.at[2], c2_v)
    pltpu.sync_copy(t4_hbm.at[3], c3_v)
    pltpu.sync_copy(p0_hbm.at[pl.ds(base, PW)], p0_v)
    pltpu.sync_copy(p1_hbm.at[pl.ds(base, PW)], p1_v)
    pltpu.sync_copy(bias_hbm, bias_v)
    bias16v = bias_v[pl.ds(0, 16)]
    bwp_s = bias16v[0]
    bep_s = bias16v[1]

    def g(gi, _):
        sl = pl.ds(gi * 16, 16)
        i0 = p0_v[sl]
        i1 = p1_v[sl]
        a0 = plsc.load_gather(c0_v, [i0])
        a2 = plsc.load_gather(c2_v, [i0])
        b1 = plsc.load_gather(c1_v, [i1])
        b3 = plsc.load_gather(c3_v, [i1])
        ew_v[sl] = jnp.maximum(a0 + b1 + bwp_s, 0.0)
        ep_v[sl] = a2 + b3 + bep_s
        return 0
    lax.fori_loop(0, PW // 16, g, 0)
    pltpu.sync_copy(ew_v, ew_hbm.at[pl.ds(base, PW)])
    pltpu.sync_copy(ep_v, ep_hbm.at[pl.ds(base, PW)])


_sc_pred = pl.kernel(
    _sc_pred_body,
    out_type=(jax.ShapeDtypeStruct((PPAD,), jnp.float32),
              jax.ShapeDtypeStruct((PPAD,), jnp.float32)),
    mesh=_mesh,
    compiler_params=pltpu.CompilerParams(needs_layout_passes=False),
    scratch_types=[
        pltpu.VMEM((NPAD,), jnp.float32),
        pltpu.VMEM((NPAD,), jnp.float32),
        pltpu.VMEM((NPAD,), jnp.float32),
        pltpu.VMEM((NPAD,), jnp.float32),
        pltpu.VMEM((PW,), jnp.int32),
        pltpu.VMEM((PW,), jnp.int32),
        pltpu.VMEM((PW,), jnp.float32),
        pltpu.VMEM((PW,), jnp.float32),
        pltpu.VMEM((16,), jnp.float32),
        pltpu.SemaphoreType.DMA,
    ],
)


@jax.jit
def _run(x, prediction_edges, message_edges, message_edgewt,
         W1p, b1p, W1f, b1f, c1, W2p, b2p, W2f, b2f, c2, Wwp, bwp, Wep, bep):
    xp = jnp.zeros((NPAD, D), jnp.float32).at[:N].set(x)
    src = message_edges[0].astype(jnp.int32)
    dst = message_edges[1].astype(jnp.int32)
    big = jnp.int32(2 ** 30)
    loop = src == dst
    tgt_all = jnp.concatenate([jnp.where(loop, big, dst),
                               jnp.where(loop, big, src)])
    src_all = jnp.concatenate([src, dst])
    s1 = 1.0 + c1 * message_edgewt
    s2 = 1.0 + c2 * message_edgewt
    s1_all = jnp.concatenate([s1, s1])
    s2_all = jnp.concatenate([s2, s2])

    y1 = _mm_y(xp, W1p.T)
    agg1 = _sc_layer(y1, tgt_all, src_all, s1_all, b1p)
    h1, y2 = _mm_h(xp, agg1, W1f[:, :D].T, W1f[:, D:].T,
                   b1f.reshape(1, D), W2p.T)
    agg2 = _sc_layer(y2, tgt_all, src_all, s2_all, b2p)

    V = jnp.zeros((D, D), jnp.float32)
    V = V.at[:, 0].set(Wwp[0, :D]).at[:, 1].set(Wwp[0, D:])
    V = V.at[:, 2].set(Wep[0, :D]).at[:, 3].set(Wep[0, D:])
    _, t = _mm_h(h1, agg2, W2f[:, :D].T, W2f[:, D:].T,
                 b2f.reshape(1, D), V)
    t4 = t[:, :4].T.copy()

    p0 = jnp.zeros((PPAD,), jnp.int32).at[:P].set(
        prediction_edges[0].astype(jnp.int32))
    p1 = jnp.zeros((PPAD,), jnp.int32).at[:P].set(
        prediction_edges[1].astype(jnp.int32))
    bias16 = jnp.zeros((16,), jnp.float32).at[0].set(bwp[0]).at[1].set(bep[0])
    ew, ep = _sc_pred(t4, p0, p1, bias16)
    return ew[:P, None], ep[:P, None]


def kernel(x, prediction_edges, message_edges, message_edgewt,
           W1p, b1p, W1f, b1f, c1, W2p, b2p, W2f, b2f, c2,
           Wwp, bwp, Wep, bep):
    return _run(x, prediction_edges, message_edges, message_edgewt,
                W1p, b1p, W1f, b1f, c1, W2p, b2p, W2f, b2f, c2,
                Wwp, bwp, Wep, bep)


# speedup vs baseline: 1.4454x; 1.2560x over previous
import jax
import jax.numpy as jnp
from jax import lax
from jax.experimental import pallas as pl
from jax.experimental.pallas import tpu as pltpu
from jax.experimental.pallas import tpu_sc as plsc

N = 10000
D = 128
E = 320000
M2E = 2 * E
P = 50000
NPAD = 10240
PPAD = 50176
NW = 32
NPT = NPAD // NW
PW = PPAD // NW
C = 1024
T = 512
SEL = T + C + 16
NEG = -3.0e38
F16 = D // 16
MW = M2E // NW
GB = 400

_mesh = plsc.VectorSubcoreMesh(core_axis_name="c", subcore_axis_name="s")


def _mm_z_body(ef_ref, s_ref, w_ref, o_ref):
    ef = ef_ref[...] * s_ref[...]
    o_ref[...] = jnp.dot(ef.astype(jnp.bfloat16), w_ref[...],
                         preferred_element_type=jnp.float32)


def _mm_z(ef, s_col, wTb):
    BM = 1024
    return pl.pallas_call(
        _mm_z_body,
        grid=(M2E // BM,),
        in_specs=[
            pl.BlockSpec((BM, D), lambda i: (i, 0)),
            pl.BlockSpec((BM, 1), lambda i: (i, 0)),
            pl.BlockSpec((D, D), lambda i: (0, 0)),
        ],
        out_specs=pl.BlockSpec((BM, D), lambda i: (i, 0)),
        out_shape=jax.ShapeDtypeStruct((M2E, D), jnp.float32),
    )(ef, s_col, wTb)


def _mm_h_body(x_ref, agg_ref, wa_ref, wb_ref, bf_ref, wn_ref, h_ref, t_ref):
    h = jnp.dot(x_ref[...].astype(jnp.bfloat16), wa_ref[...],
                preferred_element_type=jnp.float32)
    h += jnp.dot(agg_ref[...].astype(jnp.bfloat16), wb_ref[...],
                 preferred_element_type=jnp.float32)
    h = jnp.maximum(h + bf_ref[...], 0.0)
    h_ref[...] = h
    t_ref[...] = jnp.dot(h.astype(jnp.bfloat16), wn_ref[...],
                         preferred_element_type=jnp.float32)


def _mm_h(xp, agg, waTb, wbTb, bf, wnTb):
    BM = 1024
    full = pl.BlockSpec((D, D), lambda i: (0, 0))
    row = pl.BlockSpec((BM, D), lambda i: (i, 0))
    return pl.pallas_call(
        _mm_h_body,
        grid=(NPAD // BM,),
        in_specs=[row, row, full, full,
                  pl.BlockSpec((1, D), lambda i: (0, 0)), full],
        out_specs=(row, row),
        out_shape=(jax.ShapeDtypeStruct((NPAD, D), jnp.float32),
                   jax.ShapeDtypeStruct((NPAD, D), jnp.float32)),
    )(xp, agg, waTb, wbTb, bf, wnTb)


def _sc_gather_body(table_hbm, src_hbm, ef_hbm, idx_v, rows, sem):
    wid = lax.axis_index("s") * 2 + lax.axis_index("c")
    base = wid * MW

    def b(i, _):
        off = base + i * GB
        pltpu.sync_copy(src_hbm.at[pl.ds(off, GB)], idx_v)
        pltpu.async_copy(table_hbm.at[idx_v], rows, sem).wait()
        pltpu.sync_copy(rows, ef_hbm.at[pl.ds(off, GB)])
        return 0
    lax.fori_loop(0, MW // GB, b, 0)


_sc_gather = pl.kernel(
    _sc_gather_body,
    out_type=jax.ShapeDtypeStruct((M2E, D), jnp.float32),
    mesh=_mesh,
    compiler_params=pltpu.CompilerParams(needs_layout_passes=False),
    scratch_types=[
        pltpu.VMEM((GB,), jnp.int32),
        pltpu.VMEM((GB, D), jnp.float32),
        pltpu.SemaphoreType.DMA,
    ],
)


def _sc_layer_body(z_hbm, tgt_hbm, bp_hbm, agg_hbm,
                   tgt_c, sel_msg, sel_tl, rows, acc, bp_v,
                   sem, gsem):
    wid = lax.axis_index("s") * 2 + lax.axis_index("c")
    my_base = wid * NPT
    neg = jnp.full((16,), NEG, jnp.float32)
    lane = lax.iota(jnp.int32, 16)

    def init_row(i, _):
        for f in range(F16):
            acc[i, pl.ds(f * 16, 16)] = neg
        return 0
    lax.fori_loop(0, NPT + 1, init_row, 0)

    pltpu.sync_copy(bp_hbm, bp_v)

    def chunk_dma(ci, sl):
        pltpu.async_copy(tgt_hbm.at[pl.ds(ci * C, C)],
                         tgt_c.at[pl.ds(sl * C, C)], sem)

    def chunk_wait(sl):
        pltpu.make_async_copy(tgt_hbm.at[pl.ds(0, C)],
                              tgt_c.at[pl.ds(sl * C, C)], sem).wait()

    def acc_granule(m0, r0):
        tl16 = sel_tl[pl.ds(m0, 16)]
        for l in range(16):
            t_l = tl16[l]
            for f in range(F16):
                sl = pl.ds(f * 16, 16)
                acc[t_l, sl] = jnp.maximum(acc[t_l, sl], rows[r0 + l, sl])

    chunk_dma(0, 0)

    def chunk_body(ci, cursor):
        sl = ci & 1
        chunk_wait(sl)

        @pl.when(ci + 1 < M2E // C)
        def _():
            chunk_dma(ci + 1, 1 - sl)

        def gbody(g, cur):
            tg = tgt_c[pl.ds(sl * C + g * 16, 16)]
            mask = (tg >= my_base) & (tg < my_base + NPT)
            pos = plsc.cumsum(mask.astype(jnp.int32)) - 1
            idx = jnp.where(mask, cur + pos, SEL - 1)
            plsc.store_scatter(sel_msg, [idx], ci * C + g * 16 + lane)
            plsc.store_scatter(sel_tl, [idx], tg - my_base)
            return cur + jnp.max(pos) + 1
        cursor = lax.fori_loop(0, C // 16, gbody, cursor)

        def flush(cur):
            pltpu.async_copy(z_hbm.at[sel_msg.at[pl.ds(0, T)]], rows,
                             gsem).wait()

            def fbody(gi, _):
                acc_granule(gi * 16, gi * 16)
                return 0
            lax.fori_loop(0, T // 16, fbody, 0)
            nrem = cur - T
            ng = (nrem + 15) // 16

            def mv(j, _):
                dd = pl.ds(j * 16, 16)
                ss = pl.ds(T + j * 16, 16)
                sel_msg[dd] = sel_msg[ss]
                sel_tl[dd] = sel_tl[ss]
                return 0
            lax.fori_loop(0, ng, mv, 0)
            return nrem
        cursor = lax.while_loop(lambda cur: cur >= T, flush, cursor)
        return cursor

    cursor = lax.fori_loop(0, M2E // C, chunk_body, 0)

    sel_msg[pl.ds(cursor, 16)] = jnp.zeros((16,), jnp.int32)
    sel_tl[pl.ds(cursor, 16)] = jnp.full((16,), NPT, jnp.int32)
    ng = (cursor + 15) // 16

    def drain(g, _):
        pltpu.async_copy(z_hbm.at[sel_msg.at[pl.ds(g * 16, 16)]],
                         rows.at[pl.ds(0, 16)], gsem).wait()
        acc_granule(g * 16, 0)
        return 0
    lax.fori_loop(0, ng, drain, 0)

    def ep_row(i, _):
        for f in range(F16):
            sl = pl.ds(f * 16, 16)
            acc[i, sl] = jnp.maximum(acc[i, sl] + bp_v[sl], 0.0)
        return 0
    lax.fori_loop(0, NPT, ep_row, 0)
    pltpu.sync_copy(acc.at[pl.ds(0, NPT)], agg_hbm.at[pl.ds(my_base, NPT)])


_sc_layer = pl.kernel(
    _sc_layer_body,
    out_type=jax.ShapeDtypeStruct((NPAD, D), jnp.float32),
    mesh=_mesh,
    compiler_params=pltpu.CompilerParams(needs_layout_passes=False),
    scratch_types=[
        pltpu.VMEM((2 * C,), jnp.int32),
        pltpu.VMEM((SEL,), jnp.int32),
        pltpu.VMEM((SEL,), jnp.int32),
        pltpu.VMEM((T, D), jnp.float32),
        pltpu.VMEM((NPT + 1, D), jnp.float32),
        pltpu.VMEM((D,), jnp.float32),
        pltpu.SemaphoreType.DMA,
        pltpu.SemaphoreType.DMA,
    ],
)


def _sc_pred_body(t4_hbm, p0_hbm, p1_hbm, bias_hbm, ew_hbm, ep_hbm,
                  c0_v, c1_v, c2_v, c3_v, p0_v, p1_v, ew_v, ep_v, bias_v,
                  sem):
    wid = lax.axis_index("s") * 2 + lax.axis_index("c")
    base = wid * PW
    pltpu.sync_copy(t4_hbm.at[0], c0_v)
    pltpu.sync_copy(t4_hbm.at[1], c1_v)
    pltpu.sync_copy(t4_hbm.at[2], c2_v)
    pltpu.sync_copy(t4_hbm.at[3], c3_v)
    pltpu.sync_copy(p0_hbm.at[pl.ds(base, PW)], p0_v)
    pltpu.sync_copy(p1_hbm.at[pl.ds(base, PW)], p1_v)
    pltpu.sync_copy(bias_hbm, bias_v)
    bias16v = bias_v[pl.ds(0, 16)]
    bwp_s = bias16v[0]
    bep_s = bias16v[1]

    def g(gi, _):
        sl = pl.ds(gi * 16, 16)
        i0 = p0_v[sl]
        i1 = p1_v[sl]
        a0 = plsc.load_gather(c0_v, [i0])
        a2 = plsc.load_gather(c2_v, [i0])
        b1 = plsc.load_gather(c1_v, [i1])
        b3 = plsc.load_gather(c3_v, [i1])
        ew_v[sl] = jnp.maximum(a0 + b1 + bwp_s, 0.0)
        ep_v[sl] = a2 + b3 + bep_s
        return 0
    lax.fori_loop(0, PW // 16, g, 0)
    pltpu.sync_copy(ew_v, ew_hbm.at[pl.ds(base, PW)])
    pltpu.sync_copy(ep_v, ep_hbm.at[pl.ds(base, PW)])


_sc_pred = pl.kernel(
    _sc_pred_body,
    out_type=(jax.ShapeDtypeStruct((PPAD,), jnp.float32),
              jax.ShapeDtypeStruct((PPAD,), jnp.float32)),
    mesh=_mesh,
    compiler_params=pltpu.CompilerParams(needs_layout_passes=False),
    scratch_types=[
        pltpu.VMEM((NPAD,), jnp.float32),
        pltpu.VMEM((NPAD,), jnp.float32),
        pltpu.VMEM((NPAD,), jnp.float32),
        pltpu.VMEM((NPAD,), jnp.float32),
        pltpu.VMEM((PW,), jnp.int32),
        pltpu.VMEM((PW,), jnp.int32),
        pltpu.VMEM((PW,), jnp.float32),
        pltpu.VMEM((PW,), jnp.float32),
        pltpu.VMEM((16,), jnp.float32),
        pltpu.SemaphoreType.DMA,
    ],
)


@jax.jit
def _run(x, prediction_edges, message_edges, message_edgewt,
         W1p, b1p, W1f, b1f, c1, W2p, b2p, W2f, b2f, c2, Wwp, bwp, Wep, bep):
    bf16 = jnp.bfloat16
    xp = jnp.zeros((NPAD, D), jnp.float32).at[:N].set(x)
    src = message_edges[0].astype(jnp.int32)
    dst = message_edges[1].astype(jnp.int32)
    big = jnp.int32(2 ** 30)
    loop = src == dst
    tgt_all = jnp.concatenate([jnp.where(loop, big, dst),
                               jnp.where(loop, big, src)])
    src_all = jnp.concatenate([src, dst])
    s1 = 1.0 + c1 * message_edgewt
    s2 = 1.0 + c2 * message_edgewt
    s1_all = jnp.concatenate([s1, s1])[:, None]
    s2_all = jnp.concatenate([s2, s2])[:, None]

    ef1 = _sc_gather(xp, src_all)
    z1 = _mm_z(ef1, s1_all, W1p.T.astype(bf16))
    agg1 = _sc_layer(z1, tgt_all, b1p)
    h1, _ = _mm_h(xp, agg1, W1f[:, :D].T.astype(bf16),
                  W1f[:, D:].T.astype(bf16), b1f.reshape(1, D),
                  W2p.T.astype(bf16))

    ef2 = _sc_gather(h1, src_all)
    z2 = _mm_z(ef2, s2_all, W2p.T.astype(bf16))
    agg2 = _sc_layer(z2, tgt_all, b2p)

    V = jnp.zeros((D, D), jnp.float32)
    V = V.at[:, 0].set(Wwp[0, :D]).at[:, 1].set(Wwp[0, D:])
    V = V.at[:, 2].set(Wep[0, :D]).at[:, 3].set(Wep[0, D:])
    _, t = _mm_h(h1, agg2, W2f[:, :D].T.astype(bf16),
                 W2f[:, D:].T.astype(bf16), b2f.reshape(1, D),
                 V.astype(bf16))
    t4 = t[:, :4].T.copy()

    p0 = jnp.zeros((PPAD,), jnp.int32).at[:P].set(
        prediction_edges[0].astype(jnp.int32))
    p1 = jnp.zeros((PPAD,), jnp.int32).at[:P].set(
        prediction_edges[1].astype(jnp.int32))
    bias16 = jnp.zeros((16,), jnp.float32).at[0].set(bwp[0]).at[1].set(bep[0])
    ew, ep = _sc_pred(t4, p0, p1, bias16)
    return ew[:P, None], ep[:P, None]


def kernel(x, prediction_edges, message_edges, message_edgewt,
           W1p, b1p, W1f, b1f, c1, W2p, b2p, W2f, b2f, c2,
           Wwp, bwp, Wep, bep):
    return _run(x, prediction_edges, message_edges, message_edgewt,
                W1p, b1p, W1f, b1f, c1, W2p, b2p, W2f, b2f, c2,
                Wwp, bwp, Wep, bep)


# speedup vs baseline: 1.4787x; 1.0231x over previous
import jax
import jax.numpy as jnp
from jax import lax
from jax.experimental import pallas as pl
from jax.experimental.pallas import tpu as pltpu
from jax.experimental.pallas import tpu_sc as plsc

N = 10000
D = 128
E = 320000
M2E = 2 * E
P = 50000
NPAD = 10240
PPAD = 50176
NW = 32
NPT = NPAD // NW
PW = PPAD // NW
C = 1024
T = 512
SEL = T + C + 16
NEG = -3.0e38
F16 = D // 16
MW = M2E // NW
GB = 400

_mesh = plsc.VectorSubcoreMesh(core_axis_name="c", subcore_axis_name="s")


def _mm_z_body(ef_ref, s_ref, w_ref, o_ref):
    ef = ef_ref[...] * s_ref[...]
    o_ref[...] = jnp.dot(ef.astype(jnp.bfloat16), w_ref[...],
                         preferred_element_type=jnp.float32)


def _mm_z(ef, s_col, wTb):
    BM = 1024
    return pl.pallas_call(
        _mm_z_body,
        grid=(M2E // BM,),
        in_specs=[
            pl.BlockSpec((BM, D), lambda i: (i, 0)),
            pl.BlockSpec((BM, 1), lambda i: (i, 0)),
            pl.BlockSpec((D, D), lambda i: (0, 0)),
        ],
        out_specs=pl.BlockSpec((BM, D), lambda i: (i, 0)),
        out_shape=jax.ShapeDtypeStruct((M2E, D), jnp.float32),
    )(ef, s_col, wTb)


def _mm_h_body(x_ref, agg_ref, wa_ref, wb_ref, bf_ref, wn_ref, h_ref, t_ref):
    h = jnp.dot(x_ref[...].astype(jnp.bfloat16), wa_ref[...],
                preferred_element_type=jnp.float32)
    h += jnp.dot(agg_ref[...].astype(jnp.bfloat16), wb_ref[...],
                 preferred_element_type=jnp.float32)
    h = jnp.maximum(h + bf_ref[...], 0.0)
    h_ref[...] = h
    t_ref[...] = jnp.dot(h.astype(jnp.bfloat16), wn_ref[...],
                         preferred_element_type=jnp.float32)


def _mm_h(xp, agg, waTb, wbTb, bf, wnTb):
    BM = 1024
    full = pl.BlockSpec((D, D), lambda i: (0, 0))
    row = pl.BlockSpec((BM, D), lambda i: (i, 0))
    return pl.pallas_call(
        _mm_h_body,
        grid=(NPAD // BM,),
        in_specs=[row, row, full, full,
                  pl.BlockSpec((1, D), lambda i: (0, 0)), full],
        out_specs=(row, row),
        out_shape=(jax.ShapeDtypeStruct((NPAD, D), jnp.float32),
                   jax.ShapeDtypeStruct((NPAD, D), jnp.float32)),
    )(xp, agg, waTb, wbTb, bf, wnTb)


def _sc_gather_body(table_hbm, src_hbm, ef_hbm, idx_v, rows, gsem, wsem):
    wid = lax.axis_index("s") * 2 + lax.axis_index("c")
    base = wid * MW
    nb = MW // GB

    def idx_load(i, sl):
        pltpu.sync_copy(src_hbm.at[pl.ds(base + i * GB, GB)],
                        idx_v.at[pl.ds(sl * GB, GB)])

    def gather_start(sl):
        pltpu.async_copy(table_hbm.at[idx_v.at[pl.ds(sl * GB, GB)]],
                         rows.at[pl.ds(sl * GB, GB)], gsem)

    def gather_wait(sl):
        pltpu.make_async_copy(table_hbm.at[idx_v.at[pl.ds(sl * GB, GB)]],
                              rows.at[pl.ds(sl * GB, GB)], gsem).wait()

    def write_start(i, sl):
        pltpu.async_copy(rows.at[pl.ds(sl * GB, GB)],
                         ef_hbm.at[pl.ds(base + i * GB, GB)], wsem)

    def write_wait(sl):
        pltpu.make_async_copy(rows.at[pl.ds(sl * GB, GB)],
                              ef_hbm.at[pl.ds(0, GB)], wsem).wait()

    idx_load(0, 0)
    gather_start(0)

    def b(i, _):
        sl = i & 1

        @pl.when(i + 1 < nb)
        def _():
            idx_load(i + 1, 1 - sl)

            @pl.when(i >= 1)
            def _():
                write_wait(1 - sl)
            gather_start(1 - sl)
        gather_wait(sl)
        write_start(i, sl)
        return 0
    lax.fori_loop(0, nb, b, 0)
    write_wait((nb - 2) & 1)
    write_wait((nb - 1) & 1)


_sc_gather = pl.kernel(
    _sc_gather_body,
    out_type=jax.ShapeDtypeStruct((M2E, D), jnp.float32),
    mesh=_mesh,
    compiler_params=pltpu.CompilerParams(needs_layout_passes=False),
    scratch_types=[
        pltpu.VMEM((2 * GB,), jnp.int32),
        pltpu.VMEM((2 * GB, D), jnp.float32),
        pltpu.SemaphoreType.DMA,
        pltpu.SemaphoreType.DMA,
    ],
)


def _sc_layer_body(z_hbm, tgt_hbm, bp_hbm, agg_hbm,
                   tgt_c, sel_msg, sel_tl, rows, acc, bp_v,
                   sem, gsem):
    wid = lax.axis_index("s") * 2 + lax.axis_index("c")
    my_base = wid * NPT
    neg = jnp.full((16,), NEG, jnp.float32)
    lane = lax.iota(jnp.int32, 16)

    def init_row(i, _):
        for f in range(F16):
            acc[i, pl.ds(f * 16, 16)] = neg
        return 0
    lax.fori_loop(0, NPT + 1, init_row, 0)

    pltpu.sync_copy(bp_hbm, bp_v)

    def chunk_dma(ci, sl):
        pltpu.async_copy(tgt_hbm.at[pl.ds(ci * C, C)],
                         tgt_c.at[pl.ds(sl * C, C)], sem)

    def chunk_wait(sl):
        pltpu.make_async_copy(tgt_hbm.at[pl.ds(0, C)],
                              tgt_c.at[pl.ds(sl * C, C)], sem).wait()

    def acc_granule(m0, r0):
        tl16 = sel_tl[pl.ds(m0, 16)]
        for l in range(16):
            t_l = tl16[l]
            for f in range(F16):
                sl = pl.ds(f * 16, 16)
                acc[t_l, sl] = jnp.maximum(acc[t_l, sl], rows[r0 + l, sl])

    chunk_dma(0, 0)

    def chunk_body(ci, cursor):
        sl = ci & 1
        chunk_wait(sl)

        @pl.when(ci + 1 < M2E // C)
        def _():
            chunk_dma(ci + 1, 1 - sl)

        def gbody(g, cur):
            tg = tgt_c[pl.ds(sl * C + g * 16, 16)]
            mask = (tg >= my_base) & (tg < my_base + NPT)
            pos = plsc.cumsum(mask.astype(jnp.int32)) - 1
            idx = jnp.where(mask, cur + pos, SEL - 1)
            plsc.store_scatter(sel_msg, [idx], ci * C + g * 16 + lane)
            plsc.store_scatter(sel_tl, [idx], tg - my_base)
            return cur + jnp.max(pos) + 1
        cursor = lax.fori_loop(0, C // 16, gbody, cursor)

        def flush(cur):
            H = T // 2
            pltpu.async_copy(z_hbm.at[sel_msg.at[pl.ds(0, H)]],
                             rows.at[pl.ds(0, H)], gsem).wait()
            pltpu.async_copy(z_hbm.at[sel_msg.at[pl.ds(H, H)]],
                             rows.at[pl.ds(H, H)], gsem)

            def fbody(gi, _):
                acc_granule(gi * 16, gi * 16)
                return 0
            lax.fori_loop(0, H // 16, fbody, 0)
            pltpu.make_async_copy(z_hbm.at[sel_msg.at[pl.ds(H, H)]],
                                  rows.at[pl.ds(H, H)], gsem).wait()
            lax.fori_loop(H // 16, T // 16, fbody, 0)
            nrem = cur - T
            ng = (nrem + 15) // 16

            def mv(j, _):
                dd = pl.ds(j * 16, 16)
                ss = pl.ds(T + j * 16, 16)
                sel_msg[dd] = sel_msg[ss]
                sel_tl[dd] = sel_tl[ss]
                return 0
            lax.fori_loop(0, ng, mv, 0)
            return nrem
        cursor = lax.while_loop(lambda cur: cur >= T, flush, cursor)
        return cursor

    cursor = lax.fori_loop(0, M2E // C, chunk_body, 0)

    sel_msg[pl.ds(cursor, 16)] = jnp.zeros((16,), jnp.int32)
    sel_tl[pl.ds(cursor, 16)] = jnp.full((16,), NPT, jnp.int32)
    ng = (cursor + 15) // 16

    def drain(g, _):
        pltpu.async_copy(z_hbm.at[sel_msg.at[pl.ds(g * 16, 16)]],
                         rows.at[pl.ds(0, 16)], gsem).wait()
        acc_granule(g * 16, 0)
        return 0
    lax.fori_loop(0, ng, drain, 0)

    def ep_row(i, _):
        for f in range(F16):
            sl = pl.ds(f * 16, 16)
            acc[i, sl] = jnp.maximum(acc[i, sl] + bp_v[sl], 0.0)
        return 0
    lax.fori_loop(0, NPT, ep_row, 0)
    pltpu.sync_copy(acc.at[pl.ds(0, NPT)], agg_hbm.at[pl.ds(my_base, NPT)])


_sc_layer = pl.kernel(
    _sc_layer_body,
    out_type=jax.ShapeDtypeStruct((NPAD, D), jnp.float32),
    mesh=_mesh,
    compiler_params=pltpu.CompilerParams(needs_layout_passes=False),
    scratch_types=[
        pltpu.VMEM((2 * C,), jnp.int32),
        pltpu.VMEM((SEL,), jnp.int32),
        pltpu.VMEM((SEL,), jnp.int32),
        pltpu.VMEM((T, D), jnp.float32),
        pltpu.VMEM((NPT + 1, D), jnp.float32),
        pltpu.VMEM((D,), jnp.float32),
        pltpu.SemaphoreType.DMA,
        pltpu.SemaphoreType.DMA,
    ],
)


def _sc_pred_body(t4_hbm, p0_hbm, p1_hbm, bias_hbm, ew_hbm, ep_hbm,
                  c0_v, c1_v, c2_v, c3_v, p0_v, p1_v, ew_v, ep_v, bias_v,
                  sem):
    wid = lax.axis_index("s") * 2 + lax.axis_index("c")
    base = wid * PW
    pltpu.sync_copy(t4_hbm.at[0], c0_v)
    pltpu.sync_copy(t4_hbm.at[1], c1_v)
    pltpu.sync_copy(t4_hbm.at[2], c2_v)
    pltpu.sync_copy(t4_hbm.at[3], c3_v)
    pltpu.sync_copy(p0_hbm.at[pl.ds(base, PW)], p0_v)
    pltpu.sync_copy(p1_hbm.at[pl.ds(base, PW)], p1_v)
    pltpu.sync_copy(bias_hbm, bias_v)
    bias16v = bias_v[pl.ds(0, 16)]
    bwp_s = bias16v[0]
    bep_s = bias16v[1]

    def g(gi, _):
        sl = pl.ds(gi * 16, 16)
        i0 = p0_v[sl]
        i1 = p1_v[sl]
        a0 = plsc.load_gather(c0_v, [i0])
        a2 = plsc.load_gather(c2_v, [i0])
        b1 = plsc.load_gather(c1_v, [i1])
        b3 = plsc.load_gather(c3_v, [i1])
        ew_v[sl] = jnp.maximum(a0 + b1 + bwp_s, 0.0)
        ep_v[sl] = a2 + b3 + bep_s
        return 0
    lax.fori_loop(0, PW // 16, g, 0)
    pltpu.sync_copy(ew_v, ew_hbm.at[pl.ds(base, PW)])
    pltpu.sync_copy(ep_v, ep_hbm.at[pl.ds(base, PW)])


_sc_pred = pl.kernel(
    _sc_pred_body,
    out_type=(jax.ShapeDtypeStruct((PPAD,), jnp.float32),
              jax.ShapeDtypeStruct((PPAD,), jnp.float32)),
    mesh=_mesh,
    compiler_params=pltpu.CompilerParams(needs_layout_passes=False),
    scratch_types=[
        pltpu.VMEM((NPAD,), jnp.float32),
        pltpu.VMEM((NPAD,), jnp.float32),
        pltpu.VMEM((NPAD,), jnp.float32),
        pltpu.VMEM((NPAD,), jnp.float32),
        pltpu.VMEM((PW,), jnp.int32),
        pltpu.VMEM((PW,), jnp.int32),
        pltpu.VMEM((PW,), jnp.float32),
        pltpu.VMEM((PW,), jnp.float32),
        pltpu.VMEM((16,), jnp.float32),
        pltpu.SemaphoreType.DMA,
    ],
)


@jax.jit
def _run(x, prediction_edges, message_edges, message_edgewt,
         W1p, b1p, W1f, b1f, c1, W2p, b2p, W2f, b2f, c2, Wwp, bwp, Wep, bep):
    bf16 = jnp.bfloat16
    xp = jnp.zeros((NPAD, D), jnp.float32).at[:N].set(x)
    src = message_edges[0].astype(jnp.int32)
    dst = message_edges[1].astype(jnp.int32)
    big = jnp.int32(2 ** 30)
    loop = src == dst
    tgt_all = jnp.concatenate([jnp.where(loop, big, dst),
                               jnp.where(loop, big, src)])
    src_all = jnp.concatenate([src, dst])
    s1 = 1.0 + c1 * message_edgewt
    s2 = 1.0 + c2 * message_edgewt
    s1_all = jnp.concatenate([s1, s1])[:, None]
    s2_all = jnp.concatenate([s2, s2])[:, None]

    ef1 = _sc_gather(xp, src_all)
    z1 = _mm_z(ef1, s1_all, W1p.T.astype(bf16))
    agg1 = _sc_layer(z1, tgt_all, b1p)
    h1, _ = _mm_h(xp, agg1, W1f[:, :D].T.astype(bf16),
                  W1f[:, D:].T.astype(bf16), b1f.reshape(1, D),
                  W2p.T.astype(bf16))

    ef2 = _sc_gather(h1, src_all)
    z2 = _mm_z(ef2, s2_all, W2p.T.astype(bf16))
    agg2 = _sc_layer(z2, tgt_all, b2p)

    V = jnp.zeros((D, D), jnp.float32)
    V = V.at[:, 0].set(Wwp[0, :D]).at[:, 1].set(Wwp[0, D:])
    V = V.at[:, 2].set(Wep[0, :D]).at[:, 3].set(Wep[0, D:])
    _, t = _mm_h(h1, agg2, W2f[:, :D].T.astype(bf16),
                 W2f[:, D:].T.astype(bf16), b2f.reshape(1, D),
                 V.astype(bf16))
    t4 = t[:, :4].T.copy()

    p0 = jnp.zeros((PPAD,), jnp.int32).at[:P].set(
        prediction_edges[0].astype(jnp.int32))
    p1 = jnp.zeros((PPAD,), jnp.int32).at[:P].set(
        prediction_edges[1].astype(jnp.int32))
    bias16 = jnp.zeros((16,), jnp.float32).at[0].set(bwp[0]).at[1].set(bep[0])
    ew, ep = _sc_pred(t4, p0, p1, bias16)
    return ew[:P, None], ep[:P, None]


def kernel(x, prediction_edges, message_edges, message_edgewt,
           W1p, b1p, W1f, b1f, c1, W2p, b2p, W2f, b2f, c2,
           Wwp, bwp, Wep, bep):
    return _run(x, prediction_edges, message_edges, message_edgewt,
                W1p, b1p, W1f, b1f, c1, W2p, b2p, W2f, b2f, c2,
                Wwp, bwp, Wep, bep)


# speedup vs baseline: 1.5176x; 1.0263x over previous
import jax
import jax.numpy as jnp
from jax import lax
from jax.experimental import pallas as pl
from jax.experimental.pallas import tpu as pltpu
from jax.experimental.pallas import tpu_sc as plsc

N = 10000
D = 128
E = 320000
M2E = 2 * E
P = 50000
NPAD = 10240
PPAD = 50176
NW = 32
NPT = NPAD // NW
PW = PPAD // NW
C = 1024
T = 512
SEL = T + C + 16
NEG = -3.0e38
F16 = D // 16
MW = M2E // NW
GB = 400

_mesh = plsc.VectorSubcoreMesh(core_axis_name="c", subcore_axis_name="s")


def _mm_z_body(ef_ref, s_ref, w_ref, o_ref):
    ef = ef_ref[...] * s_ref[...]
    o_ref[...] = jnp.dot(ef.astype(jnp.bfloat16), w_ref[...],
                         preferred_element_type=jnp.float32)


def _mm_z(ef, s_col, wTb):
    BM = 1024
    return pl.pallas_call(
        _mm_z_body,
        grid=(M2E // BM,),
        in_specs=[
            pl.BlockSpec((BM, D), lambda i: (i, 0)),
            pl.BlockSpec((BM, 1), lambda i: (i, 0)),
            pl.BlockSpec((D, D), lambda i: (0, 0)),
        ],
        out_specs=pl.BlockSpec((BM, D), lambda i: (i, 0)),
        out_shape=jax.ShapeDtypeStruct((M2E, D), jnp.float32),
    )(ef, s_col, wTb)


def _mm_h_body(x_ref, agg_ref, wa_ref, wb_ref, bf_ref, wn_ref, h_ref, t_ref):
    h = jnp.dot(x_ref[...].astype(jnp.bfloat16), wa_ref[...],
                preferred_element_type=jnp.float32)
    h += jnp.dot(agg_ref[...].astype(jnp.bfloat16), wb_ref[...],
                 preferred_element_type=jnp.float32)
    h = jnp.maximum(h + bf_ref[...], 0.0)
    h_ref[...] = h
    t_ref[...] = jnp.dot(h.astype(jnp.bfloat16), wn_ref[...],
                         preferred_element_type=jnp.float32)


def _mm_h(xp, agg, waTb, wbTb, bf, wnTb):
    BM = 1024
    full = pl.BlockSpec((D, D), lambda i: (0, 0))
    row = pl.BlockSpec((BM, D), lambda i: (i, 0))
    return pl.pallas_call(
        _mm_h_body,
        grid=(NPAD // BM,),
        in_specs=[row, row, full, full,
                  pl.BlockSpec((1, D), lambda i: (0, 0)), full],
        out_specs=(row, row),
        out_shape=(jax.ShapeDtypeStruct((NPAD, D), jnp.float32),
                   jax.ShapeDtypeStruct((NPAD, D), jnp.float32)),
    )(xp, agg, waTb, wbTb, bf, wnTb)


def _sc_gather_body(table_hbm, src_hbm, ef_hbm, idx_v, rows, gsem, wsem):
    wid = lax.axis_index("s") * 2 + lax.axis_index("c")
    base = wid * MW
    nb = MW // GB

    def idx_load(i, sl):
        pltpu.sync_copy(src_hbm.at[pl.ds(base + i * GB, GB)],
                        idx_v.at[pl.ds(sl * GB, GB)])

    def gather_start(sl):
        pltpu.async_copy(table_hbm.at[idx_v.at[pl.ds(sl * GB, GB)]],
                         rows.at[pl.ds(sl * GB, GB)], gsem)

    def gather_wait(sl):
        pltpu.make_async_copy(table_hbm.at[idx_v.at[pl.ds(sl * GB, GB)]],
                              rows.at[pl.ds(sl * GB, GB)], gsem).wait()

    def write_start(i, sl):
        pltpu.async_copy(rows.at[pl.ds(sl * GB, GB)],
                         ef_hbm.at[pl.ds(base + i * GB, GB)], wsem)

    def write_wait(sl):
        pltpu.make_async_copy(rows.at[pl.ds(sl * GB, GB)],
                              ef_hbm.at[pl.ds(0, GB)], wsem).wait()

    idx_load(0, 0)
    gather_start(0)

    def b(i, _):
        sl = i & 1

        @pl.when(i + 1 < nb)
        def _():
            idx_load(i + 1, 1 - sl)

            @pl.when(i >= 1)
            def _():
                write_wait(1 - sl)
            gather_start(1 - sl)
        gather_wait(sl)
        write_start(i, sl)
        return 0
    lax.fori_loop(0, nb, b, 0)
    write_wait((nb - 2) & 1)
    write_wait((nb - 1) & 1)


_sc_gather = pl.kernel(
    _sc_gather_body,
    out_type=jax.ShapeDtypeStruct((M2E, D), jnp.float32),
    mesh=_mesh,
    compiler_params=pltpu.CompilerParams(needs_layout_passes=False),
    scratch_types=[
        pltpu.VMEM((2 * GB,), jnp.int32),
        pltpu.VMEM((2 * GB, D), jnp.float32),
        pltpu.SemaphoreType.DMA,
        pltpu.SemaphoreType.DMA,
    ],
)


def _sc_layer_body(z_hbm, tgt_hbm, bp_hbm, agg_hbm,
                   tgt_c, sel_msg, sel_tl, rows, acc, bp_v,
                   sem, gsem):
    wid = lax.axis_index("s") * 2 + lax.axis_index("c")
    my_base = wid * NPT
    neg = jnp.full((16,), NEG, jnp.float32)
    lane = lax.iota(jnp.int32, 16)

    def init_row(i, _):
        for f in range(F16):
            acc[i, pl.ds(f * 16, 16)] = neg
        return 0
    lax.fori_loop(0, NPT + 1, init_row, 0)

    pltpu.sync_copy(bp_hbm, bp_v)

    def chunk_dma(ci, sl):
        pltpu.async_copy(tgt_hbm.at[pl.ds(ci * C, C)],
                         tgt_c.at[pl.ds(sl * C, C)], sem)

    def chunk_wait(sl):
        pltpu.make_async_copy(tgt_hbm.at[pl.ds(0, C)],
                              tgt_c.at[pl.ds(sl * C, C)], sem).wait()

    def acc_granule(m0, r0):
        tl16 = sel_tl[pl.ds(m0, 16)]
        for l in range(16):
            t_l = tl16[l]
            for f in range(F16):
                sl = pl.ds(f * 16, 16)
                acc[t_l, sl] = jnp.maximum(acc[t_l, sl], rows[r0 + l, sl])

    chunk_dma(0, 0)

    def chunk_body(ci, cursor):
        sl = ci & 1
        chunk_wait(sl)

        @pl.when(ci + 1 < M2E // C)
        def _():
            chunk_dma(ci + 1, 1 - sl)

        def gbody(g, cur):
            tg = tgt_c[pl.ds(sl * C + g * 16, 16)]
            mask = (tg >= my_base) & (tg < my_base + NPT)
            pos = plsc.cumsum(mask.astype(jnp.int32)) - 1
            idx = jnp.where(mask, cur + pos, SEL - 1)
            plsc.store_scatter(sel_msg, [idx], ci * C + g * 16 + lane)
            plsc.store_scatter(sel_tl, [idx], tg - my_base)
            cnt = plsc.all_reduce_population_count(mask)[0]
            return cur + cnt
        cursor = lax.fori_loop(0, C // 16, gbody, cursor, unroll=4)

        def flush(cur):
            H = T // 2
            pltpu.async_copy(z_hbm.at[sel_msg.at[pl.ds(0, H)]],
                             rows.at[pl.ds(0, H)], gsem).wait()
            pltpu.async_copy(z_hbm.at[sel_msg.at[pl.ds(H, H)]],
                             rows.at[pl.ds(H, H)], gsem)

            def fbody(gi, _):
                acc_granule(gi * 16, gi * 16)
                return 0
            lax.fori_loop(0, H // 16, fbody, 0)
            pltpu.make_async_copy(z_hbm.at[sel_msg.at[pl.ds(H, H)]],
                                  rows.at[pl.ds(H, H)], gsem).wait()
            lax.fori_loop(H // 16, T // 16, fbody, 0)
            nrem = cur - T
            ng = (nrem + 15) // 16

            def mv(j, _):
                dd = pl.ds(j * 16, 16)
                ss = pl.ds(T + j * 16, 16)
                sel_msg[dd] = sel_msg[ss]
                sel_tl[dd] = sel_tl[ss]
                return 0
            lax.fori_loop(0, ng, mv, 0)
            return nrem
        cursor = lax.while_loop(lambda cur: cur >= T, flush, cursor)
        return cursor

    cursor = lax.fori_loop(0, M2E // C, chunk_body, 0)

    sel_msg[pl.ds(cursor, 16)] = jnp.zeros((16,), jnp.int32)
    sel_tl[pl.ds(cursor, 16)] = jnp.full((16,), NPT, jnp.int32)
    ng = (cursor + 15) // 16

    def drain(g, _):
        pltpu.async_copy(z_hbm.at[sel_msg.at[pl.ds(g * 16, 16)]],
                         rows.at[pl.ds(0, 16)], gsem).wait()
        acc_granule(g * 16, 0)
        return 0
    lax.fori_loop(0, ng, drain, 0)

    def ep_row(i, _):
        for f in range(F16):
            sl = pl.ds(f * 16, 16)
            acc[i, sl] = jnp.maximum(acc[i, sl] + bp_v[sl], 0.0)
        return 0
    lax.fori_loop(0, NPT, ep_row, 0)
    pltpu.sync_copy(acc.at[pl.ds(0, NPT)], agg_hbm.at[pl.ds(my_base, NPT)])


_sc_layer = pl.kernel(
    _sc_layer_body,
    out_type=jax.ShapeDtypeStruct((NPAD, D), jnp.float32),
    mesh=_mesh,
    compiler_params=pltpu.CompilerParams(needs_layout_passes=False),
    scratch_types=[
        pltpu.VMEM((2 * C,), jnp.int32),
        pltpu.VMEM((SEL,), jnp.int32),
        pltpu.VMEM((SEL,), jnp.int32),
        pltpu.VMEM((T, D), jnp.float32),
        pltpu.VMEM((NPT + 1, D), jnp.float32),
        pltpu.VMEM((D,), jnp.float32),
        pltpu.SemaphoreType.DMA,
        pltpu.SemaphoreType.DMA,
    ],
)


def _sc_pred_body(t4_hbm, p0_hbm, p1_hbm, bias_hbm, ew_hbm, ep_hbm,
                  c0_v, c1_v, c2_v, c3_v, p0_v, p1_v, ew_v, ep_v, bias_v,
                  sem):
    wid = lax.axis_index("s") * 2 + lax.axis_index("c")
    base = wid * PW
    pltpu.sync_copy(t4_hbm.at[0], c0_v)
    pltpu.sync_copy(t4_hbm.at[1], c1_v)
    pltpu.sync_copy(t4_hbm.at[2], c2_v)
    pltpu.sync_copy(t4_hbm.at[3], c3_v)
    pltpu.sync_copy(p0_hbm.at[pl.ds(base, PW)], p0_v)
    pltpu.sync_copy(p1_hbm.at[pl.ds(base, PW)], p1_v)
    pltpu.sync_copy(bias_hbm, bias_v)
    bias16v = bias_v[pl.ds(0, 16)]
    bwp_s = bias16v[0]
    bep_s = bias16v[1]

    def g(gi, _):
        sl = pl.ds(gi * 16, 16)
        i0 = p0_v[sl]
        i1 = p1_v[sl]
        a0 = plsc.load_gather(c0_v, [i0])
        a2 = plsc.load_gather(c2_v, [i0])
        b1 = plsc.load_gather(c1_v, [i1])
        b3 = plsc.load_gather(c3_v, [i1])
        ew_v[sl] = jnp.maximum(a0 + b1 + bwp_s, 0.0)
        ep_v[sl] = a2 + b3 + bep_s
        return 0
    lax.fori_loop(0, PW // 16, g, 0)
    pltpu.sync_copy(ew_v, ew_hbm.at[pl.ds(base, PW)])
    pltpu.sync_copy(ep_v, ep_hbm.at[pl.ds(base, PW)])


_sc_pred = pl.kernel(
    _sc_pred_body,
    out_type=(jax.ShapeDtypeStruct((PPAD,), jnp.float32),
              jax.ShapeDtypeStruct((PPAD,), jnp.float32)),
    mesh=_mesh,
    compiler_params=pltpu.CompilerParams(needs_layout_passes=False),
    scratch_types=[
        pltpu.VMEM((NPAD,), jnp.float32),
        pltpu.VMEM((NPAD,), jnp.float32),
        pltpu.VMEM((NPAD,), jnp.float32),
        pltpu.VMEM((NPAD,), jnp.float32),
        pltpu.VMEM((PW,), jnp.int32),
        pltpu.VMEM((PW,), jnp.int32),
        pltpu.VMEM((PW,), jnp.float32),
        pltpu.VMEM((PW,), jnp.float32),
        pltpu.VMEM((16,), jnp.float32),
        pltpu.SemaphoreType.DMA,
    ],
)


@jax.jit
def _run(x, prediction_edges, message_edges, message_edgewt,
         W1p, b1p, W1f, b1f, c1, W2p, b2p, W2f, b2f, c2, Wwp, bwp, Wep, bep):
    bf16 = jnp.bfloat16
    xp = jnp.zeros((NPAD, D), jnp.float32).at[:N].set(x)
    src = message_edges[0].astype(jnp.int32)
    dst = message_edges[1].astype(jnp.int32)
    big = jnp.int32(2 ** 30)
    loop = src == dst
    tgt_all = jnp.concatenate([jnp.where(loop, big, dst),
                               jnp.where(loop, big, src)])
    src_all = jnp.concatenate([src, dst])
    s1 = 1.0 + c1 * message_edgewt
    s2 = 1.0 + c2 * message_edgewt
    s1_all = jnp.concatenate([s1, s1])[:, None]
    s2_all = jnp.concatenate([s2, s2])[:, None]

    ef1 = _sc_gather(xp, src_all)
    z1 = _mm_z(ef1, s1_all, W1p.T.astype(bf16))
    agg1 = _sc_layer(z1, tgt_all, b1p)
    h1, _ = _mm_h(xp, agg1, W1f[:, :D].T.astype(bf16),
                  W1f[:, D:].T.astype(bf16), b1f.reshape(1, D),
                  W2p.T.astype(bf16))

    ef2 = _sc_gather(h1, src_all)
    z2 = _mm_z(ef2, s2_all, W2p.T.astype(bf16))
    agg2 = _sc_layer(z2, tgt_all, b2p)

    V = jnp.zeros((D, D), jnp.float32)
    V = V.at[:, 0].set(Wwp[0, :D]).at[:, 1].set(Wwp[0, D:])
    V = V.at[:, 2].set(Wep[0, :D]).at[:, 3].set(Wep[0, D:])
    _, t = _mm_h(h1, agg2, W2f[:, :D].T.astype(bf16),
                 W2f[:, D:].T.astype(bf16), b2f.reshape(1, D),
                 V.astype(bf16))
    t4 = t[:, :4].T.copy()

    p0 = jnp.zeros((PPAD,), jnp.int32).at[:P].set(
        prediction_edges[0].astype(jnp.int32))
    p1 = jnp.zeros((PPAD,), jnp.int32).at[:P].set(
        prediction_edges[1].astype(jnp.int32))
    bias16 = jnp.zeros((16,), jnp.float32).at[0].set(bwp[0]).at[1].set(bep[0])
    ew, ep = _sc_pred(t4, p0, p1, bias16)
    return ew[:P, None], ep[:P, None]


def kernel(x, prediction_edges, message_edges, message_edgewt,
           W1p, b1p, W1f, b1f, c1, W2p, b2p, W2f, b2f, c2,
           Wwp, bwp, Wep, bep):
    return _run(x, prediction_edges, message_edges, message_edgewt,
                W1p, b1p, W1f, b1f, c1, W2p, b2p, W2f, b2f, c2,
                Wwp, bwp, Wep, bep)


# speedup vs baseline: 1.5297x; 1.0080x over previous
import jax
import jax.numpy as jnp
from jax import lax
from jax.experimental import pallas as pl
from jax.experimental.pallas import tpu as pltpu
from jax.experimental.pallas import tpu_sc as plsc

N = 10000
D = 128
E = 320000
M2E = 2 * E
P = 50000
NPAD = 10240
PPAD = 50176
NW = 32
NPT = NPAD // NW
PW = PPAD // NW
C = 2000
T = 512
SEL = T + C + 16
NEG = -3.0e38
F16 = D // 16
MW = M2E // NW
GB = 400

_mesh = plsc.VectorSubcoreMesh(core_axis_name="c", subcore_axis_name="s")


def _mm_z_body(ef_ref, s_ref, w_ref, o_ref):
    ef = ef_ref[...] * s_ref[...]
    o_ref[...] = jnp.dot(ef.astype(jnp.bfloat16), w_ref[...],
                         preferred_element_type=jnp.float32)


def _mm_z(ef, s_col, wTb):
    BM = 1024
    return pl.pallas_call(
        _mm_z_body,
        grid=(M2E // BM,),
        in_specs=[
            pl.BlockSpec((BM, D), lambda i: (i, 0)),
            pl.BlockSpec((BM, 1), lambda i: (i, 0)),
            pl.BlockSpec((D, D), lambda i: (0, 0)),
        ],
        out_specs=pl.BlockSpec((BM, D), lambda i: (i, 0)),
        out_shape=jax.ShapeDtypeStruct((M2E, D), jnp.float32),
    )(ef, s_col, wTb)


def _mm_h_body(x_ref, agg_ref, wa_ref, wb_ref, bf_ref, wn_ref, h_ref, t_ref):
    h = jnp.dot(x_ref[...].astype(jnp.bfloat16), wa_ref[...],
                preferred_element_type=jnp.float32)
    h += jnp.dot(agg_ref[...].astype(jnp.bfloat16), wb_ref[...],
                 preferred_element_type=jnp.float32)
    h = jnp.maximum(h + bf_ref[...], 0.0)
    h_ref[...] = h
    t_ref[...] = jnp.dot(h.astype(jnp.bfloat16), wn_ref[...],
                         preferred_element_type=jnp.float32)


def _mm_h(xp, agg, waTb, wbTb, bf, wnTb):
    BM = 1024
    full = pl.BlockSpec((D, D), lambda i: (0, 0))
    row = pl.BlockSpec((BM, D), lambda i: (i, 0))
    return pl.pallas_call(
        _mm_h_body,
        grid=(NPAD // BM,),
        in_specs=[row, row, full, full,
                  pl.BlockSpec((1, D), lambda i: (0, 0)), full],
        out_specs=(row, row),
        out_shape=(jax.ShapeDtypeStruct((NPAD, D), jnp.float32),
                   jax.ShapeDtypeStruct((NPAD, D), jnp.float32)),
    )(xp, agg, waTb, wbTb, bf, wnTb)


def _sc_gather_body(table_hbm, src_hbm, ef_hbm, idx_v, rows, gsem, wsem):
    wid = lax.axis_index("s") * 2 + lax.axis_index("c")
    base = wid * MW
    nb = MW // GB

    def idx_load(i, sl):
        pltpu.sync_copy(src_hbm.at[pl.ds(base + i * GB, GB)],
                        idx_v.at[pl.ds(sl * GB, GB)])

    def gather_start(sl):
        pltpu.async_copy(table_hbm.at[idx_v.at[pl.ds(sl * GB, GB)]],
                         rows.at[pl.ds(sl * GB, GB)], gsem)

    def gather_wait(sl):
        pltpu.make_async_copy(table_hbm.at[idx_v.at[pl.ds(sl * GB, GB)]],
                              rows.at[pl.ds(sl * GB, GB)], gsem).wait()

    def write_start(i, sl):
        pltpu.async_copy(rows.at[pl.ds(sl * GB, GB)],
                         ef_hbm.at[pl.ds(base + i * GB, GB)], wsem)

    def write_wait(sl):
        pltpu.make_async_copy(rows.at[pl.ds(sl * GB, GB)],
                              ef_hbm.at[pl.ds(0, GB)], wsem).wait()

    idx_load(0, 0)
    gather_start(0)

    def b(i, _):
        sl = i & 1

        @pl.when(i + 1 < nb)
        def _():
            idx_load(i + 1, 1 - sl)

            @pl.when(i >= 1)
            def _():
                write_wait(1 - sl)
            gather_start(1 - sl)
        gather_wait(sl)
        write_start(i, sl)
        return 0
    lax.fori_loop(0, nb, b, 0)
    write_wait((nb - 2) & 1)
    write_wait((nb - 1) & 1)


_sc_gather = pl.kernel(
    _sc_gather_body,
    out_type=jax.ShapeDtypeStruct((M2E, D), jnp.float32),
    mesh=_mesh,
    compiler_params=pltpu.CompilerParams(needs_layout_passes=False),
    scratch_types=[
        pltpu.VMEM((2 * GB,), jnp.int32),
        pltpu.VMEM((2 * GB, D), jnp.float32),
        pltpu.SemaphoreType.DMA,
        pltpu.SemaphoreType.DMA,
    ],
)


def _sc_layer_body(z_hbm, tgt_hbm, bp_hbm, agg_hbm,
                   tgt_c, sel_msg, sel_tl, rows, acc, bp_v,
                   sem, gsem):
    wid = lax.axis_index("s") * 2 + lax.axis_index("c")
    my_base = wid * NPT
    neg = jnp.full((16,), NEG, jnp.float32)
    lane = lax.iota(jnp.int32, 16)

    def init_row(i, _):
        for f in range(F16):
            acc[i, pl.ds(f * 16, 16)] = neg
        return 0
    lax.fori_loop(0, NPT + 1, init_row, 0)

    pltpu.sync_copy(bp_hbm, bp_v)

    def chunk_dma(ci, sl):
        pltpu.async_copy(tgt_hbm.at[pl.ds(ci * C, C)],
                         tgt_c.at[pl.ds(sl * C, C)], sem)

    def chunk_wait(sl):
        pltpu.make_async_copy(tgt_hbm.at[pl.ds(0, C)],
                              tgt_c.at[pl.ds(sl * C, C)], sem).wait()

    def acc_granule(m0, r0):
        tl16 = sel_tl[pl.ds(m0, 16)]
        for l in range(16):
            t_l = tl16[l]
            for f in range(F16):
                sl = pl.ds(f * 16, 16)
                acc[t_l, sl] = jnp.maximum(acc[t_l, sl], rows[r0 + l, sl])

    chunk_dma(0, 0)

    def chunk_body(ci, cursor):
        sl = ci & 1
        chunk_wait(sl)

        @pl.when(ci + 1 < M2E // C)
        def _():
            chunk_dma(ci + 1, 1 - sl)

        def gbody(g, cur):
            tg = tgt_c[pl.ds(sl * C + g * 16, 16)]
            mask = (tg >= my_base) & (tg < my_base + NPT)
            pos = plsc.cumsum(mask.astype(jnp.int32)) - 1
            idx = jnp.where(mask, cur + pos, SEL - 1)
            plsc.store_scatter(sel_msg, [idx], ci * C + g * 16 + lane)
            plsc.store_scatter(sel_tl, [idx], tg - my_base)
            cnt = plsc.all_reduce_population_count(mask)[0]
            return cur + cnt
        cursor = lax.fori_loop(0, C // 16, gbody, cursor, unroll=4)

        def flush(cur):
            H = T // 2
            pltpu.async_copy(z_hbm.at[sel_msg.at[pl.ds(0, H)]],
                             rows.at[pl.ds(0, H)], gsem).wait()
            pltpu.async_copy(z_hbm.at[sel_msg.at[pl.ds(H, H)]],
                             rows.at[pl.ds(H, H)], gsem)

            def fbody(gi, _):
                acc_granule(gi * 16, gi * 16)
                return 0
            lax.fori_loop(0, H // 16, fbody, 0)
            pltpu.make_async_copy(z_hbm.at[sel_msg.at[pl.ds(H, H)]],
                                  rows.at[pl.ds(H, H)], gsem).wait()
            lax.fori_loop(H // 16, T // 16, fbody, 0)
            nrem = cur - T
            ng = (nrem + 15) // 16

            def mv(j, _):
                dd = pl.ds(j * 16, 16)
                ss = pl.ds(T + j * 16, 16)
                sel_msg[dd] = sel_msg[ss]
                sel_tl[dd] = sel_tl[ss]
                return 0
            lax.fori_loop(0, ng, mv, 0)
            return nrem
        cursor = lax.while_loop(lambda cur: cur >= T, flush, cursor)
        return cursor

    cursor = lax.fori_loop(0, M2E // C, chunk_body, 0)

    sel_msg[pl.ds(cursor, 16)] = jnp.zeros((16,), jnp.int32)
    sel_tl[pl.ds(cursor, 16)] = jnp.full((16,), NPT, jnp.int32)
    ng = (cursor + 15) // 16

    def drain(g, _):
        pltpu.async_copy(z_hbm.at[sel_msg.at[pl.ds(g * 16, 16)]],
                         rows.at[pl.ds(0, 16)], gsem).wait()
        acc_granule(g * 16, 0)
        return 0
    lax.fori_loop(0, ng, drain, 0)

    def ep_row(i, _):
        for f in range(F16):
            sl = pl.ds(f * 16, 16)
            acc[i, sl] = jnp.maximum(acc[i, sl] + bp_v[sl], 0.0)
        return 0
    lax.fori_loop(0, NPT, ep_row, 0)
    pltpu.sync_copy(acc.at[pl.ds(0, NPT)], agg_hbm.at[pl.ds(my_base, NPT)])


_sc_layer = pl.kernel(
    _sc_layer_body,
    out_type=jax.ShapeDtypeStruct((NPAD, D), jnp.float32),
    mesh=_mesh,
    compiler_params=pltpu.CompilerParams(needs_layout_passes=False),
    scratch_types=[
        pltpu.VMEM((2 * C,), jnp.int32),
        pltpu.VMEM((SEL,), jnp.int32),
        pltpu.VMEM((SEL,), jnp.int32),
        pltpu.VMEM((T, D), jnp.float32),
        pltpu.VMEM((NPT + 1, D), jnp.float32),
        pltpu.VMEM((D,), jnp.float32),
        pltpu.SemaphoreType.DMA,
        pltpu.SemaphoreType.DMA,
    ],
)


def _sc_pred_body(t4_hbm, p0_hbm, p1_hbm, bias_hbm, ew_hbm, ep_hbm,
                  c0_v, c1_v, c2_v, c3_v, p0_v, p1_v, ew_v, ep_v, bias_v,
                  sem):
    wid = lax.axis_index("s") * 2 + lax.axis_index("c")
    base = wid * PW
    pltpu.sync_copy(t4_hbm.at[0], c0_v)
    pltpu.sync_copy(t4_hbm.at[1], c1_v)
    pltpu.sync_copy(t4_hbm.at[2], c2_v)
    pltpu.sync_copy(t4_hbm.at[3], c3_v)
    pltpu.sync_copy(p0_hbm.at[pl.ds(base, PW)], p0_v)
    pltpu.sync_copy(p1_hbm.at[pl.ds(base, PW)], p1_v)
    pltpu.sync_copy(bias_hbm, bias_v)
    bias16v = bias_v[pl.ds(0, 16)]
    bwp_s = bias16v[0]
    bep_s = bias16v[1]

    def g(gi, _):
        sl = pl.ds(gi * 16, 16)
        i0 = p0_v[sl]
        i1 = p1_v[sl]
        a0 = plsc.load_gather(c0_v, [i0])
        a2 = plsc.load_gather(c2_v, [i0])
        b1 = plsc.load_gather(c1_v, [i1])
        b3 = plsc.load_gather(c3_v, [i1])
        ew_v[sl] = jnp.maximum(a0 + b1 + bwp_s, 0.0)
        ep_v[sl] = a2 + b3 + bep_s
        return 0
    lax.fori_loop(0, PW // 16, g, 0)
    pltpu.sync_copy(ew_v, ew_hbm.at[pl.ds(base, PW)])
    pltpu.sync_copy(ep_v, ep_hbm.at[pl.ds(base, PW)])


_sc_pred = pl.kernel(
    _sc_pred_body,
    out_type=(jax.ShapeDtypeStruct((PPAD,), jnp.float32),
              jax.ShapeDtypeStruct((PPAD,), jnp.float32)),
    mesh=_mesh,
    compiler_params=pltpu.CompilerParams(needs_layout_passes=False),
    scratch_types=[
        pltpu.VMEM((NPAD,), jnp.float32),
        pltpu.VMEM((NPAD,), jnp.float32),
        pltpu.VMEM((NPAD,), jnp.float32),
        pltpu.VMEM((NPAD,), jnp.float32),
        pltpu.VMEM((PW,), jnp.int32),
        pltpu.VMEM((PW,), jnp.int32),
        pltpu.VMEM((PW,), jnp.float32),
        pltpu.VMEM((PW,), jnp.float32),
        pltpu.VMEM((16,), jnp.float32),
        pltpu.SemaphoreType.DMA,
    ],
)


@jax.jit
def _run(x, prediction_edges, message_edges, message_edgewt,
         W1p, b1p, W1f, b1f, c1, W2p, b2p, W2f, b2f, c2, Wwp, bwp, Wep, bep):
    bf16 = jnp.bfloat16
    xp = jnp.zeros((NPAD, D), jnp.float32).at[:N].set(x)
    src = message_edges[0].astype(jnp.int32)
    dst = message_edges[1].astype(jnp.int32)
    big = jnp.int32(2 ** 30)
    loop = src == dst
    tgt_all = jnp.concatenate([jnp.where(loop, big, dst),
                               jnp.where(loop, big, src)])
    src_all = jnp.concatenate([src, dst])
    s1 = 1.0 + c1 * message_edgewt
    s2 = 1.0 + c2 * message_edgewt
    s1_all = jnp.concatenate([s1, s1])[:, None]
    s2_all = jnp.concatenate([s2, s2])[:, None]

    ef1 = _sc_gather(xp, src_all)
    z1 = _mm_z(ef1, s1_all, W1p.T.astype(bf16))
    agg1 = _sc_layer(z1, tgt_all, b1p)
    h1, _ = _mm_h(xp, agg1, W1f[:, :D].T.astype(bf16),
                  W1f[:, D:].T.astype(bf16), b1f.reshape(1, D),
                  W2p.T.astype(bf16))

    ef2 = _sc_gather(h1, src_all)
    z2 = _mm_z(ef2, s2_all, W2p.T.astype(bf16))
    agg2 = _sc_layer(z2, tgt_all, b2p)

    V = jnp.zeros((D, D), jnp.float32)
    V = V.at[:, 0].set(Wwp[0, :D]).at[:, 1].set(Wwp[0, D:])
    V = V.at[:, 2].set(Wep[0, :D]).at[:, 3].set(Wep[0, D:])
    _, t = _mm_h(h1, agg2, W2f[:, :D].T.astype(bf16),
                 W2f[:, D:].T.astype(bf16), b2f.reshape(1, D),
                 V.astype(bf16))
    t4 = t[:, :4].T.copy()

    p0 = jnp.zeros((PPAD,), jnp.int32).at[:P].set(
        prediction_edges[0].astype(jnp.int32))
    p1 = jnp.zeros((PPAD,), jnp.int32).at[:P].set(
        prediction_edges[1].astype(jnp.int32))
    bias16 = jnp.zeros((16,), jnp.float32).at[0].set(bwp[0]).at[1].set(bep[0])
    ew, ep = _sc_pred(t4, p0, p1, bias16)
    return ew[:P, None], ep[:P, None]


def kernel(x, prediction_edges, message_edges, message_edgewt,
           W1p, b1p, W1f, b1f, c1, W2p, b2p, W2f, b2f, c2,
           Wwp, bwp, Wep, bep):
    return _run(x, prediction_edges, message_edges, message_edgewt,
                W1p, b1p, W1f, b1f, c1, W2p, b2p, W2f, b2f, c2,
                Wwp, bwp, Wep, bep)
